# baseline (speedup 1.0000x reference)
import jax
import jax.numpy as jnp
from jax import lax
from jax.experimental import pallas as pl
from jax.experimental.pallas import tpu as pltpu
from jax.experimental.pallas import tpu_sc as plsc

N = 10000
E = 320000
D = 128
H = 256

NC = 2
NS = 16
NT = NC * NS

EH = E // 2
PT = EH // NT

ACH = 40
ANCH = 2 * EH // NT // ACH

CCH = 40
CNCH = PT // CCH
PT1 = E // NT
DCH = 80
DNCH = PT1 // DCH
SEG = 64
CPAD = 128

BN = 2000
BE = 1600

F32 = jnp.float32
BF16 = jnp.bfloat16


def _mlp(a, w1, b1, w2, b2):
    h = jnp.dot(a.astype(BF16), w1, preferred_element_type=F32) + b1
    h = jnp.maximum(h, 0.0)
    return jnp.dot(h.astype(BF16), w2, preferred_element_type=F32) + b2


def _inorm(h):
    m = jnp.mean(h, axis=-1, keepdims=True)
    c = h - m
    v = jnp.mean(c * c, axis=-1, keepdims=True)
    return c * lax.rsqrt(v + 1e-5)


def _full(shape):
    return pl.BlockSpec(shape, lambda i: tuple(0 for _ in shape))


def _mesh():
    return plsc.VectorSubcoreMesh(core_axis_name="c", subcore_axis_name="s")


def _tc_pre_body(x_ref, g_ref,
                 n1w1, n1b1, n1w2, n1b2,
                 n2w1, n2b1, n2w2, n2b2,
                 e3w1, e3b1, e3w2, e3b2,
                 n3w1, n3b1, n3w2, n3b2,
                 msg_ref, hx_ref, gvec_ref):
    i = pl.program_id(0)
    x = x_ref[...]
    hx_ref[...] = _mlp(x, n1w1[...], n1b1[...], n1w2[...], n1b2[...])
    msg_ref[...] = _mlp(x, n2w1[...], n2b1[...], n2w2[...], n2b2[...])

    @pl.when(i == 0)
    def _():
        g = g_ref[...]
        gvec_ref[...] = jnp.zeros((8, D), F32)
        gvec_ref[0:1, :] = _mlp(g, e3w1[...], e3b1[...], e3w2[...], e3b2[...])
        gvec_ref[1:2, :] = _mlp(g, n3w1[...], n3b1[...], n3w2[...], n3b2[...])


def _tc_pre(x, g, wts):
    wspecs = [_full(w.shape) for w in wts]
    return pl.pallas_call(
        _tc_pre_body,
        grid=(N // BN,),
        in_specs=[pl.BlockSpec((BN, D), lambda i: (i, 0)), _full((1, D))] + wspecs,
        out_specs=[pl.BlockSpec((BN, D), lambda i: (i, 0)),
                   pl.BlockSpec((BN, D), lambda i: (i, 0)),
                   _full((8, D))],
        out_shape=[jax.ShapeDtypeStruct((N, D), F32),
                   jax.ShapeDtypeStruct((N, D), F32),
                   jax.ShapeDtypeStruct((8, D), F32)],
    )(x, g, *wts)


def _sc_gather_body(x_hbm, idx3_hbm, out_hbm,
                    idx_all, r0, r1, g0, g1, o0, o1):
    wid = lax.axis_index("c") * NS + lax.axis_index("s")
    pltpu.sync_copy(idx3_hbm.at[wid], idx_all)

    def gissue(c, buf, sem):
        return pltpu.async_copy(x_hbm.at[idx_all.at[c]], buf, sem)

    def oissue(c, buf, sem):
        base = wid * (ANCH * ACH) + c * ACH
        return pltpu.async_copy(buf, out_hbm.at[pl.ds(base, ACH)], sem)

    gissue(0, r0, g0)
    gissue(1, r1, g1)

    @pl.loop(0, ANCH // 2)
    def _(it):
        c = it * 2
        pltpu.make_async_copy(x_hbm.at[idx_all.at[c]], r0, g0).wait()
        h0 = oissue(c, r0, o0)
        pltpu.make_async_copy(x_hbm.at[idx_all.at[c + 1]], r1, g1).wait()
        h1 = oissue(c + 1, r1, o1)
        h0.wait()
        h1.wait()

        @pl.when(it < ANCH // 2 - 1)
        def _():
            gissue(c + 2, r0, g0)
            gissue(c + 3, r1, g1)


def _sc_gather(x, idx3):
    fn = pl.kernel(
        _sc_gather_body,
        mesh=_mesh(),
        out_type=jax.ShapeDtypeStruct((2 * EH, D), F32),
        scratch_types=[pltpu.VMEM((ANCH, ACH), jnp.int32),
                       pltpu.VMEM((ACH, D), F32),
                       pltpu.VMEM((ACH, D), F32),
                       pltpu.SemaphoreType.DMA,
                       pltpu.SemaphoreType.DMA,
                       pltpu.SemaphoreType.DMA,
                       pltpu.SemaphoreType.DMA],
    )
    return fn(x, idx3)


def _tc_edge_body(xs_ref, xd_ref, e_ref, gvec_ref,
                  e1w1, e1b1, e1w2, e1b2,
                  e2w1, e2b1, e2w2, e2b2,
                  enew_ref, s_ref, esum_ref):
    i = pl.program_id(0)
    ns = xs_ref[...] + xd_ref[...]
    e = e_ref[...]
    he = (_mlp(ns, e1w1[...], e1b1[...], e1w2[...], e1b2[...])
          + _mlp(e, e2w1[...], e2b1[...], e2w2[...], e2b2[...])
          + gvec_ref[0:1, :])
    en = e + jnp.maximum(_inorm(he), 0.0)
    enew_ref[...] = en
    s_ref[...] = 1.0 / (1.0 + jnp.exp(-en))

    @pl.when(i == 0)
    def _():
        esum_ref[...] = jnp.zeros((8, D), F32)

    esum_ref[0:1, :] += jnp.sum(en, axis=0, keepdims=True)


def _tc_edge(xsd, e, gvec, wts, half):
    wspecs = [_full(w.shape) for w in wts]
    nb = EH // BE
    off = half * nb
    return pl.pallas_call(
        _tc_edge_body,
        grid=(nb,),
        in_specs=[pl.BlockSpec((BE, D), lambda i: (i, 0)),
                  pl.BlockSpec((BE, D), lambda i, _nb=nb: (i + _nb, 0)),
                  pl.BlockSpec((BE, D), lambda i, _o=off: (i + _o, 0)),
                  _full((8, D))] + wspecs,
        out_specs=[pl.BlockSpec((BE, D), lambda i: (i, 0)),
                   pl.BlockSpec((BE, D), lambda i: (i, 0)),
                   _full((8, D))],
        out_shape=[jax.ShapeDtypeStruct((EH, D), F32),
                   jax.ShapeDtypeStruct((EH, D), F32),
                   jax.ShapeDtypeStruct((8, D), F32)],
    )(xsd, xsd, e, gvec, *wts)


def _sc_denom_body(s1_hbm, s2_hbm, src3_hbm, dst3_hbm, z_hbm, out_hbm,
                   isa, ida, r0, r1, acc_sh, l0, l1, a0, a1):
    cid = lax.axis_index("c")
    sid = lax.axis_index("s")
    tid = cid * NS + sid
    ebase = sid * PT1

    rb = sid * 640

    @pl.when(sid < NS - 1)
    def _():
        pltpu.sync_copy(z_hbm.at[pl.ds(rb, 640)], acc_sh.at[pl.ds(rb, 640)])

    @pl.when(sid == NS - 1)
    def _():
        pltpu.sync_copy(z_hbm.at[pl.ds(9600, 400)], acc_sh.at[pl.ds(9600, 400)])

    plsc.subcore_barrier()

    def lissue(c, buf, sem):
        sl = pl.ds(ebase + c * DCH, DCH)

        @pl.when(cid == 0)
        def _():
            pltpu.async_copy(s1_hbm.at[sl], buf, sem)

        @pl.when(cid == 1)
        def _():
            pltpu.async_copy(s2_hbm.at[sl], buf, sem)

    def lwait(buf, sem):
        pltpu.make_async_copy(s1_hbm.at[pl.ds(ebase, DCH)], buf, sem).wait()

    lissue(0, r0, l0)
    lissue(1, r1, l1)

    @pl.loop(0, DNCH // 2)
    def _(it):
        c = it * 2

        @pl.when((c & (SEG - 1)) == 0)
        def _():
            cs = pl.multiple_of(c, SEG)
            pltpu.sync_copy(src3_hbm.at[tid, pl.ds(cs, SEG)], isa)
            pltpu.sync_copy(dst3_hbm.at[tid, pl.ds(cs, SEG)], ida)

        r = c & (SEG - 1)
        lwait(r0, l0)
        h0 = pltpu.async_copy(r0, acc_sh.at[isa.at[r]], a0, add=True)
        h1 = pltpu.async_copy(r0, acc_sh.at[ida.at[r]], a0, add=True)
        lwait(r1, l1)
        h2 = pltpu.async_copy(r1, acc_sh.at[isa.at[r + 1]], a1, add=True)
        h3 = pltpu.async_copy(r1, acc_sh.at[ida.at[r + 1]], a1, add=True)
        h0.wait()
        h1.wait()
        h2.wait()
        h3.wait()

        @pl.when(it < DNCH // 2 - 1)
        def _():
            lissue(c + 2, r0, l0)
            lissue(c + 3, r1, l1)

        @pl.when(it == DNCH // 2 - 1)
        def _():
            lissue(DNCH - 1, r0, l0)

    rt = (DNCH - 1) & (SEG - 1)
    lwait(r0, l0)
    ht0 = pltpu.async_copy(r0, acc_sh.at[isa.at[rt]], a0, add=True)
    ht1 = pltpu.async_copy(r0, acc_sh.at[ida.at[rt]], a0, add=True)
    ht0.wait()
    ht1.wait()

    plsc.subcore_barrier()

    @pl.when(sid < NS - 1)
    def _():
        pltpu.sync_copy(acc_sh.at[pl.ds(rb, 640)], out_hbm.at[cid, pl.ds(rb, 640)])

    @pl.when(sid == NS - 1)
    def _():
        pltpu.sync_copy(acc_sh.at[pl.ds(9600, 400)],
                        out_hbm.at[cid, pl.ds(9600, 400)])


def _sc_denom(s1, s2, src3, dst3, zeros_nd):
    fn = pl.kernel(
        _sc_denom_body,
        mesh=_mesh(),
        out_type=jax.ShapeDtypeStruct((NC, N, D), F32),
        scratch_types=[pltpu.VMEM((SEG, DCH), jnp.int32),
                       pltpu.VMEM((SEG, DCH), jnp.int32),
                       pltpu.VMEM((DCH, D), F32),
                       pltpu.VMEM((DCH, D), F32),
                       pltpu.VMEM_SHARED((N, D), F32),
                       pltpu.SemaphoreType.DMA,
                       pltpu.SemaphoreType.DMA,
                       pltpu.SemaphoreType.DMA,
                       pltpu.SemaphoreType.DMA],
    )
    return fn(s1, s2, src3, dst3, zeros_nd)


def _sc_agg_body(s_hbm, src3_hbm, dst3_hbm, msg_hbm, z_hbm, out_hbm,
                 isa, ida, s0, s1, md0, md1, ms0, ms1, acc_sh,
                 l0, l1, g0, g1, a0, a1):
    cid = lax.axis_index("c")
    sid = lax.axis_index("s")
    tid = cid * NS + sid
    ebase = tid * PT

    rb = sid * 640

    @pl.when(sid < NS - 1)
    def _():
        pltpu.sync_copy(z_hbm.at[pl.ds(rb, 640)], acc_sh.at[pl.ds(rb, 640)])

    @pl.when(sid == NS - 1)
    def _():
        pltpu.sync_copy(z_hbm.at[pl.ds(9600, 400)], acc_sh.at[pl.ds(9600, 400)])

    plsc.subcore_barrier()

    def fill(r, c, sbuf, mdbuf, msbuf, lsem, gsem):
        pltpu.async_copy(s_hbm.at[pl.ds(ebase + c * CCH, CCH)], sbuf, lsem)
        pltpu.async_copy(msg_hbm.at[ida.at[r]], mdbuf, gsem)
        pltpu.async_copy(msg_hbm.at[isa.at[r]], msbuf, gsem)

    def mult(sbuf, mdbuf, msbuf):
        @pl.loop(0, CCH, step=2)
        def _(r):
            for rr in range(2):
                for j in range(0, D, 16):
                    sv = sbuf[r + rr, pl.ds(j, 16)]
                    mdbuf[r + rr, pl.ds(j, 16)] = mdbuf[r + rr, pl.ds(j, 16)] * sv
                    msbuf[r + rr, pl.ds(j, 16)] = msbuf[r + rr, pl.ds(j, 16)] * sv

    def drain(sbuf, mdbuf, msbuf, lsem, gsem):
        pltpu.make_async_copy(s_hbm.at[pl.ds(ebase, CCH)], sbuf, lsem).wait()
        pltpu.make_async_copy(msg_hbm.at[isa.at[0]], mdbuf, gsem).wait()
        pltpu.make_async_copy(msg_hbm.at[isa.at[0]], msbuf, gsem).wait()

    pltpu.sync_copy(src3_hbm.at[tid, pl.ds(0, SEG)], isa)
    pltpu.sync_copy(dst3_hbm.at[tid, pl.ds(0, SEG)], ida)
    fill(0, 0, s0, md0, ms0, l0, g0)
    fill(1, 1, s1, md1, ms1, l1, g1)

    @pl.loop(0, CNCH // 2)
    def _(it):
        c = it * 2
        r = c & (SEG - 1)
        drain(s0, md0, ms0, l0, g0)
        mult(s0, md0, ms0)
        h0 = pltpu.async_copy(md0, acc_sh.at[isa.at[r]], a0, add=True)
        h1 = pltpu.async_copy(ms0, acc_sh.at[ida.at[r]], a0, add=True)

        drain(s1, md1, ms1, l1, g1)
        mult(s1, md1, ms1)
        h2 = pltpu.async_copy(md1, acc_sh.at[isa.at[r + 1]], a1, add=True)
        h3 = pltpu.async_copy(ms1, acc_sh.at[ida.at[r + 1]], a1, add=True)

        h0.wait()
        h1.wait()
        h2.wait()
        h3.wait()

        @pl.when((((c + 2) & (SEG - 1)) == 0) & (it < CNCH // 2))
        def _():
            cs = pl.multiple_of(c + 2, SEG)
            pltpu.sync_copy(src3_hbm.at[tid, pl.ds(cs, SEG)], isa)
            pltpu.sync_copy(dst3_hbm.at[tid, pl.ds(cs, SEG)], ida)

        @pl.when(it < CNCH // 2 - 1)
        def _():
            fill((c + 2) & (SEG - 1), c + 2, s0, md0, ms0, l0, g0)
            fill((c + 3) & (SEG - 1), c + 3, s1, md1, ms1, l1, g1)

        @pl.when(it == CNCH // 2 - 1)
        def _():
            fill((CNCH - 1) & (SEG - 1), CNCH - 1, s0, md0, ms0, l0, g0)

    rt = (CNCH - 1) & (SEG - 1)
    drain(s0, md0, ms0, l0, g0)
    mult(s0, md0, ms0)
    ht0 = pltpu.async_copy(md0, acc_sh.at[isa.at[rt]], a0, add=True)
    ht1 = pltpu.async_copy(ms0, acc_sh.at[ida.at[rt]], a0, add=True)
    ht0.wait()
    ht1.wait()

    plsc.subcore_barrier()

    @pl.when(sid < NS - 1)
    def _():
        pltpu.sync_copy(acc_sh.at[pl.ds(rb, 640)], out_hbm.at[cid, pl.ds(rb, 640)])

    @pl.when(sid == NS - 1)
    def _():
        pltpu.sync_copy(acc_sh.at[pl.ds(9600, 400)],
                        out_hbm.at[cid, pl.ds(9600, 400)])


def _sc_agg(s, src3, dst3, msg, zeros_nd):
    fn = pl.kernel(
        _sc_agg_body,
        mesh=_mesh(),
        out_type=jax.ShapeDtypeStruct((NC, N, D), F32),
        scratch_types=[pltpu.VMEM((SEG, CCH), jnp.int32),
                       pltpu.VMEM((SEG, CCH), jnp.int32),
                       pltpu.VMEM((CCH, D), F32),
                       pltpu.VMEM((CCH, D), F32),
                       pltpu.VMEM((CCH, D), F32),
                       pltpu.VMEM((CCH, D), F32),
                       pltpu.VMEM((CCH, D), F32),
                       pltpu.VMEM((CCH, D), F32),
                       pltpu.VMEM_SHARED((N, D), F32),
                       pltpu.SemaphoreType.DMA,
                       pltpu.SemaphoreType.DMA,
                       pltpu.SemaphoreType.DMA,
                       pltpu.SemaphoreType.DMA,
                       pltpu.SemaphoreType.DMA,
                       pltpu.SemaphoreType.DMA],
    )
    return fn(s, src3, dst3, msg, zeros_nd)


def _tc_node_body(x_ref, hx_ref, dp_ref, ap1_ref, ap2_ref,
                  gvec_ref, es1_ref, es2_ref, g_ref,
                  g1w1, g1b1, g1w2, g1b2,
                  g2w1, g2b1, g2w2, g2b2,
                  g3w1, g3b1, g3w2, g3b2,
                  xnew_ref, gnew_ref, xsum_ref):
    i = pl.program_id(0)
    d = dp_ref[0] + dp_ref[1] + 1e-7
    a = (ap1_ref[0] + ap1_ref[1]) + (ap2_ref[0] + ap2_ref[1])
    hn = hx_ref[...] + a / d + gvec_ref[1:2, :]
    xn = x_ref[...] + jnp.maximum(_inorm(hn), 0.0)
    xnew_ref[...] = xn

    @pl.when(i == 0)
    def _():
        xsum_ref[...] = jnp.zeros((8, D), F32)
        gnew_ref[...] = jnp.zeros((8, D), F32)

    xsum_ref[0:1, :] += jnp.sum(xn, axis=0, keepdims=True)

    @pl.when(i == (N // BN) - 1)
    def _():
        g = g_ref[...]
        xm = xsum_ref[0:1, :] * (1.0 / N)
        em = (es1_ref[0:1, :] + es2_ref[0:1, :]) * (1.0 / E)
        hg = (_mlp(xm, g1w1[...], g1b1[...], g1w2[...], g1b2[...])
              + _mlp(em, g2w1[...], g2b1[...], g2w2[...], g2b2[...])
              + _mlp(g, g3w1[...], g3b1[...], g3w2[...], g3b2[...]))
        gnew_ref[0:1, :] = g + jnp.maximum(_inorm(hg), 0.0)


def _tc_node(x, hx, dp, ap1, ap2, gvec, es1, es2, g, wts):
    wspecs = [_full(w.shape) for w in wts]
    part = pl.BlockSpec((NC, BN, D), lambda i: (0, i, 0))
    return pl.pallas_call(
        _tc_node_body,
        grid=(N // BN,),
        in_specs=[pl.BlockSpec((BN, D), lambda i: (i, 0)),
                  pl.BlockSpec((BN, D), lambda i: (i, 0)),
                  part, part, part,
                  _full((8, D)), _full((8, D)), _full((8, D)),
                  _full((1, D))] + wspecs,
        out_specs=[pl.BlockSpec((BN, D), lambda i: (i, 0)),
                   _full((8, D))],
        out_shape=[jax.ShapeDtypeStruct((N, D), F32),
                   jax.ShapeDtypeStruct((8, D), F32)],
        scratch_shapes=[pltpu.VMEM((8, D), F32)],
    )(x, hx, dp, ap1, ap2, gvec, es1, es2, g, *wts)


def _wt(p):
    w1, b1, w2, b2 = p
    return (w1.T.astype(BF16), b1.reshape(1, H),
            w2.T.astype(BF16), b2.reshape(1, D))


def _idx3(v, nch, chunk):
    v3 = v.reshape(NT, nch, chunk)
    pad = jnp.zeros((NT, CPAD - nch, chunk), jnp.int32)
    return jnp.concatenate([v3, pad], axis=1)


def _idx3d(v):
    v3 = v.reshape(NT, DNCH, DCH)
    pad = jnp.zeros((NT, CPAD - DNCH, DCH), jnp.int32)
    return jnp.concatenate([v3, pad], axis=1)


def kernel(node_features, edge_index, edge_features, global_features, params):
    x = node_features[0]
    e = edge_features[0]
    g = global_features[0]
    src = edge_index[0, 0]
    dst = edge_index[0, 1]
    p = params

    pre_wts = _wt(p["node1"]) + _wt(p["node2"]) + _wt(p["edge3"]) + _wt(p["node3"])
    msg, hx, gvec = _tc_pre(x, g, pre_wts)

    edge_wts = _wt(p["edge1"]) + _wt(p["edge2"])
    zeros_nd = jnp.zeros((N, D), F32)

    results = []
    for h in (0, 1):
        src_h = lax.slice_in_dim(src, h * EH, (h + 1) * EH)
        dst_h = lax.slice_in_dim(dst, h * EH, (h + 1) * EH)
        sd3 = jnp.concatenate([src_h, dst_h]).reshape(NT, ANCH, ACH)
        src3 = _idx3(src_h, CNCH, CCH)
        dst3 = _idx3(dst_h, CNCH, CCH)

        xsd = _sc_gather(x, sd3)
        e_new_h, s_h, esum_h = _tc_edge(xsd, e, gvec, edge_wts, h)
        apart_h = _sc_agg(s_h, src3, dst3, msg, zeros_nd)
        results.append((e_new_h, s_h, esum_h, apart_h))

    (e_new1, s1, es1, ap1), (e_new2, s2, es2, ap2) = results

    src3f = _idx3d(src)
    dst3f = _idx3d(dst)
    dp = _sc_denom(s1, s2, src3f, dst3f, zeros_nd)

    glob_wts = _wt(p["glob1"]) + _wt(p["glob2"]) + _wt(p["glob3"])
    x_new, g8 = _tc_node(x, hx, dp, ap1, ap2, gvec, es1, es2, g, glob_wts)

    e_new = jnp.concatenate([e_new1, e_new2], axis=0)
    return (x_new[None], e_new[None], g8[0:1][None])


# speedup vs baseline: 1.0761x; 1.0761x over previous
import jax
import jax.numpy as jnp
from jax import lax
from jax.experimental import pallas as pl
from jax.experimental.pallas import tpu as pltpu
from jax.experimental.pallas import tpu_sc as plsc

N = 10000
E = 320000
D = 128
H = 256

NC = 2
NS = 16
NT = NC * NS

EH = E // 2
PT = EH // NT

ACH = 40
ANCH = 2 * EH // NT // ACH

CCH = 40
CNCH = PT // CCH
PT1 = E // NT
DCH = 80
DNCH = PT1 // DCH
SEG = 64
CPAD = 128

BN = 2000
BE = 1600

F32 = jnp.float32
BF16 = jnp.bfloat16


def _mlp(a, w1, b1, w2, b2):
    h = jnp.dot(a.astype(BF16), w1, preferred_element_type=F32) + b1
    h = jnp.maximum(h, 0.0)
    return jnp.dot(h.astype(BF16), w2, preferred_element_type=F32) + b2


def _inorm(h):
    m = jnp.mean(h, axis=-1, keepdims=True)
    c = h - m
    v = jnp.mean(c * c, axis=-1, keepdims=True)
    return c * lax.rsqrt(v + 1e-5)


def _full(shape):
    return pl.BlockSpec(shape, lambda i: tuple(0 for _ in shape))


def _mesh():
    return plsc.VectorSubcoreMesh(core_axis_name="c", subcore_axis_name="s")


def _tc_pre_body(x_ref, g_ref,
                 n1w1, n1b1, n1w2, n1b2,
                 n2w1, n2b1, n2w2, n2b2,
                 e3w1, e3b1, e3w2, e3b2,
                 n3w1, n3b1, n3w2, n3b2,
                 msg_ref, hx_ref, gvec_ref):
    i = pl.program_id(0)
    x = x_ref[...]
    hx_ref[...] = _mlp(x, n1w1[...], n1b1[...], n1w2[...], n1b2[...])
    msg_ref[...] = _mlp(x, n2w1[...], n2b1[...], n2w2[...], n2b2[...])

    @pl.when(i == 0)
    def _():
        g = g_ref[...]
        gvec_ref[...] = jnp.zeros((8, D), F32)
        gvec_ref[0:1, :] = _mlp(g, e3w1[...], e3b1[...], e3w2[...], e3b2[...])
        gvec_ref[1:2, :] = _mlp(g, n3w1[...], n3b1[...], n3w2[...], n3b2[...])


def _tc_pre(x, g, wts):
    wspecs = [_full(w.shape) for w in wts]
    return pl.pallas_call(
        _tc_pre_body,
        grid=(N // BN,),
        in_specs=[pl.BlockSpec((BN, D), lambda i: (i, 0)), _full((1, D))] + wspecs,
        out_specs=[pl.BlockSpec((BN, D), lambda i: (i, 0)),
                   pl.BlockSpec((BN, D), lambda i: (i, 0)),
                   _full((8, D))],
        out_shape=[jax.ShapeDtypeStruct((N, D), F32),
                   jax.ShapeDtypeStruct((N, D), F32),
                   jax.ShapeDtypeStruct((8, D), F32)],
    )(x, g, *wts)


def _sc_gather_body(x_hbm, idx3_hbm, out_hbm,
                    idx_all, r0, r1, g0, g1, o0, o1):
    wid = lax.axis_index("c") * NS + lax.axis_index("s")
    pltpu.sync_copy(idx3_hbm.at[wid], idx_all)

    def gissue(c, buf, sem):
        return pltpu.async_copy(x_hbm.at[idx_all.at[c]], buf, sem)

    def oissue(c, buf, sem):
        base = wid * (ANCH * ACH) + c * ACH
        return pltpu.async_copy(buf, out_hbm.at[pl.ds(base, ACH)], sem)

    gissue(0, r0, g0)
    gissue(1, r1, g1)

    @pl.loop(0, ANCH // 2)
    def _(it):
        c = it * 2
        pltpu.make_async_copy(x_hbm.at[idx_all.at[c]], r0, g0).wait()
        h0 = oissue(c, r0, o0)
        pltpu.make_async_copy(x_hbm.at[idx_all.at[c + 1]], r1, g1).wait()
        h1 = oissue(c + 1, r1, o1)
        h0.wait()
        h1.wait()

        @pl.when(it < ANCH // 2 - 1)
        def _():
            gissue(c + 2, r0, g0)
            gissue(c + 3, r1, g1)


def _sc_gather(x, idx3):
    fn = pl.kernel(
        _sc_gather_body,
        mesh=_mesh(),
        out_type=jax.ShapeDtypeStruct((2 * EH, D), F32),
        scratch_types=[pltpu.VMEM((ANCH, ACH), jnp.int32),
                       pltpu.VMEM((ACH, D), F32),
                       pltpu.VMEM((ACH, D), F32),
                       pltpu.SemaphoreType.DMA,
                       pltpu.SemaphoreType.DMA,
                       pltpu.SemaphoreType.DMA,
                       pltpu.SemaphoreType.DMA],
    )
    return fn(x, idx3)


def _tc_edge_body(xs_ref, xd_ref, e_ref, gvec_ref,
                  e1w1, e1b1, e1w2, e1b2,
                  e2w1, e2b1, e2w2, e2b2,
                  enew_ref, s_ref, esum_ref):
    i = pl.program_id(0)
    ns = xs_ref[...] + xd_ref[...]
    e = e_ref[...]
    he = (_mlp(ns, e1w1[...], e1b1[...], e1w2[...], e1b2[...])
          + _mlp(e, e2w1[...], e2b1[...], e2w2[...], e2b2[...])
          + gvec_ref[0:1, :])
    en = e + jnp.maximum(_inorm(he), 0.0)
    enew_ref[...] = en
    s_ref[...] = 1.0 / (1.0 + jnp.exp(-en))

    @pl.when(i == 0)
    def _():
        esum_ref[...] = jnp.zeros((8, D), F32)

    esum_ref[0:1, :] += jnp.sum(en, axis=0, keepdims=True)


def _tc_edge(xsd, e, gvec, wts, half):
    wspecs = [_full(w.shape) for w in wts]
    nb = EH // BE
    off = half * nb
    return pl.pallas_call(
        _tc_edge_body,
        grid=(nb,),
        in_specs=[pl.BlockSpec((BE, D), lambda i: (i, 0)),
                  pl.BlockSpec((BE, D), lambda i, _nb=nb: (i + _nb, 0)),
                  pl.BlockSpec((BE, D), lambda i, _o=off: (i + _o, 0)),
                  _full((8, D))] + wspecs,
        out_specs=[pl.BlockSpec((BE, D), lambda i: (i, 0)),
                   pl.BlockSpec((BE, D), lambda i: (i, 0)),
                   _full((8, D))],
        out_shape=[jax.ShapeDtypeStruct((EH, D), F32),
                   jax.ShapeDtypeStruct((EH, D), F32),
                   jax.ShapeDtypeStruct((8, D), F32)],
    )(xsd, xsd, e, gvec, *wts)


def _sc_denom_body(s_hbm, src3_hbm, dst3_hbm, z_hbm, out_hbm,
                   isa, ida, r0, r1, acc_sh, l0, l1, a0, a1):
    cid = lax.axis_index("c")
    sid = lax.axis_index("s")
    tid = cid * NS + sid
    ebase = tid * PT

    rb = sid * 640

    @pl.when(sid < NS - 1)
    def _():
        pltpu.sync_copy(z_hbm.at[pl.ds(rb, 640)], acc_sh.at[pl.ds(rb, 640)])

    @pl.when(sid == NS - 1)
    def _():
        pltpu.sync_copy(z_hbm.at[pl.ds(9600, 400)], acc_sh.at[pl.ds(9600, 400)])

    plsc.subcore_barrier()

    def lissue(c, buf, sem):
        return pltpu.async_copy(s_hbm.at[pl.ds(ebase + c * CCH, CCH)], buf, sem)

    lissue(0, r0, l0)
    lissue(1, r1, l1)

    @pl.loop(0, CNCH // 2)
    def _(it):
        c = it * 2

        @pl.when((c & (SEG - 1)) == 0)
        def _():
            cs = pl.multiple_of(c, SEG)
            pltpu.sync_copy(src3_hbm.at[tid, pl.ds(cs, SEG)], isa)
            pltpu.sync_copy(dst3_hbm.at[tid, pl.ds(cs, SEG)], ida)

        r = c & (SEG - 1)
        pltpu.make_async_copy(s_hbm.at[pl.ds(ebase, CCH)], r0, l0).wait()
        h0 = pltpu.async_copy(r0, acc_sh.at[isa.at[r]], a0, add=True)
        h1 = pltpu.async_copy(r0, acc_sh.at[ida.at[r]], a0, add=True)
        pltpu.make_async_copy(s_hbm.at[pl.ds(ebase, CCH)], r1, l1).wait()
        h2 = pltpu.async_copy(r1, acc_sh.at[isa.at[r + 1]], a1, add=True)
        h3 = pltpu.async_copy(r1, acc_sh.at[ida.at[r + 1]], a1, add=True)
        h0.wait()
        h1.wait()
        h2.wait()
        h3.wait()

        @pl.when(it < CNCH // 2 - 1)
        def _():
            lissue(c + 2, r0, l0)
            lissue(c + 3, r1, l1)

        @pl.when(it == CNCH // 2 - 1)
        def _():
            lissue(CNCH - 1, r0, l0)

    rt = (CNCH - 1) & (SEG - 1)
    pltpu.make_async_copy(s_hbm.at[pl.ds(ebase, CCH)], r0, l0).wait()
    ht0 = pltpu.async_copy(r0, acc_sh.at[isa.at[rt]], a0, add=True)
    ht1 = pltpu.async_copy(r0, acc_sh.at[ida.at[rt]], a0, add=True)
    ht0.wait()
    ht1.wait()

    plsc.subcore_barrier()

    @pl.when(sid < NS - 1)
    def _():
        pltpu.sync_copy(acc_sh.at[pl.ds(rb, 640)], out_hbm.at[cid, pl.ds(rb, 640)])

    @pl.when(sid == NS - 1)
    def _():
        pltpu.sync_copy(acc_sh.at[pl.ds(9600, 400)],
                        out_hbm.at[cid, pl.ds(9600, 400)])


def _sc_denom(s, src3, dst3, zeros_nd):
    fn = pl.kernel(
        _sc_denom_body,
        mesh=_mesh(),
        out_type=jax.ShapeDtypeStruct((NC, N, D), F32),
        scratch_types=[pltpu.VMEM((SEG, CCH), jnp.int32),
                       pltpu.VMEM((SEG, CCH), jnp.int32),
                       pltpu.VMEM((CCH, D), F32),
                       pltpu.VMEM((CCH, D), F32),
                       pltpu.VMEM_SHARED((N, D), F32),
                       pltpu.SemaphoreType.DMA,
                       pltpu.SemaphoreType.DMA,
                       pltpu.SemaphoreType.DMA,
                       pltpu.SemaphoreType.DMA],
    )
    return fn(s, src3, dst3, zeros_nd)


def _sc_agg_body(s_hbm, src3_hbm, dst3_hbm, msg_hbm, z_hbm, out_hbm,
                 isa, ida, s0, s1, md0, md1, ms0, ms1, acc_sh,
                 l0, l1, g0, g1, a0, a1):
    cid = lax.axis_index("c")
    sid = lax.axis_index("s")
    tid = cid * NS + sid
    ebase = tid * PT

    rb = sid * 640

    @pl.when(sid < NS - 1)
    def _():
        pltpu.sync_copy(z_hbm.at[pl.ds(rb, 640)], acc_sh.at[pl.ds(rb, 640)])

    @pl.when(sid == NS - 1)
    def _():
        pltpu.sync_copy(z_hbm.at[pl.ds(9600, 400)], acc_sh.at[pl.ds(9600, 400)])

    plsc.subcore_barrier()

    def fill(r, c, sbuf, mdbuf, msbuf, lsem, gsem):
        pltpu.async_copy(s_hbm.at[pl.ds(ebase + c * CCH, CCH)], sbuf, lsem)
        pltpu.async_copy(msg_hbm.at[ida.at[r]], mdbuf, gsem)
        pltpu.async_copy(msg_hbm.at[isa.at[r]], msbuf, gsem)

    def mult(sbuf, mdbuf, msbuf):
        @pl.loop(0, CCH, step=2)
        def _(r):
            for rr in range(2):
                for j in range(0, D, 16):
                    sv = sbuf[r + rr, pl.ds(j, 16)]
                    mdbuf[r + rr, pl.ds(j, 16)] = mdbuf[r + rr, pl.ds(j, 16)] * sv
                    msbuf[r + rr, pl.ds(j, 16)] = msbuf[r + rr, pl.ds(j, 16)] * sv

    def drain(sbuf, mdbuf, msbuf, lsem, gsem):
        pltpu.make_async_copy(s_hbm.at[pl.ds(ebase, CCH)], sbuf, lsem).wait()
        pltpu.make_async_copy(msg_hbm.at[isa.at[0]], mdbuf, gsem).wait()
        pltpu.make_async_copy(msg_hbm.at[isa.at[0]], msbuf, gsem).wait()

    pltpu.sync_copy(src3_hbm.at[tid, pl.ds(0, SEG)], isa)
    pltpu.sync_copy(dst3_hbm.at[tid, pl.ds(0, SEG)], ida)
    fill(0, 0, s0, md0, ms0, l0, g0)
    fill(1, 1, s1, md1, ms1, l1, g1)

    @pl.loop(0, CNCH // 2)
    def _(it):
        c = it * 2
        r = c & (SEG - 1)
        drain(s0, md0, ms0, l0, g0)
        mult(s0, md0, ms0)
        h0 = pltpu.async_copy(md0, acc_sh.at[isa.at[r]], a0, add=True)
        h1 = pltpu.async_copy(ms0, acc_sh.at[ida.at[r]], a0, add=True)

        drain(s1, md1, ms1, l1, g1)
        mult(s1, md1, ms1)
        h2 = pltpu.async_copy(md1, acc_sh.at[isa.at[r + 1]], a1, add=True)
        h3 = pltpu.async_copy(ms1, acc_sh.at[ida.at[r + 1]], a1, add=True)

        h0.wait()
        h1.wait()
        h2.wait()
        h3.wait()

        @pl.when((((c + 2) & (SEG - 1)) == 0) & (it < CNCH // 2))
        def _():
            cs = pl.multiple_of(c + 2, SEG)
            pltpu.sync_copy(src3_hbm.at[tid, pl.ds(cs, SEG)], isa)
            pltpu.sync_copy(dst3_hbm.at[tid, pl.ds(cs, SEG)], ida)

        @pl.when(it < CNCH // 2 - 1)
        def _():
            fill((c + 2) & (SEG - 1), c + 2, s0, md0, ms0, l0, g0)
            fill((c + 3) & (SEG - 1), c + 3, s1, md1, ms1, l1, g1)

        @pl.when(it == CNCH // 2 - 1)
        def _():
            fill((CNCH - 1) & (SEG - 1), CNCH - 1, s0, md0, ms0, l0, g0)

    rt = (CNCH - 1) & (SEG - 1)
    drain(s0, md0, ms0, l0, g0)
    mult(s0, md0, ms0)
    ht0 = pltpu.async_copy(md0, acc_sh.at[isa.at[rt]], a0, add=True)
    ht1 = pltpu.async_copy(ms0, acc_sh.at[ida.at[rt]], a0, add=True)
    ht0.wait()
    ht1.wait()

    plsc.subcore_barrier()

    @pl.when(sid < NS - 1)
    def _():
        pltpu.sync_copy(acc_sh.at[pl.ds(rb, 640)], out_hbm.at[cid, pl.ds(rb, 640)])

    @pl.when(sid == NS - 1)
    def _():
        pltpu.sync_copy(acc_sh.at[pl.ds(9600, 400)],
                        out_hbm.at[cid, pl.ds(9600, 400)])


def _sc_agg(s, src3, dst3, msg, zeros_nd):
    fn = pl.kernel(
        _sc_agg_body,
        mesh=_mesh(),
        out_type=jax.ShapeDtypeStruct((NC, N, D), F32),
        scratch_types=[pltpu.VMEM((SEG, CCH), jnp.int32),
                       pltpu.VMEM((SEG, CCH), jnp.int32),
                       pltpu.VMEM((CCH, D), F32),
                       pltpu.VMEM((CCH, D), F32),
                       pltpu.VMEM((CCH, D), F32),
                       pltpu.VMEM((CCH, D), F32),
                       pltpu.VMEM((CCH, D), F32),
                       pltpu.VMEM((CCH, D), F32),
                       pltpu.VMEM_SHARED((N, D), F32),
                       pltpu.SemaphoreType.DMA,
                       pltpu.SemaphoreType.DMA,
                       pltpu.SemaphoreType.DMA,
                       pltpu.SemaphoreType.DMA,
                       pltpu.SemaphoreType.DMA,
                       pltpu.SemaphoreType.DMA],
    )
    return fn(s, src3, dst3, msg, zeros_nd)


def _tc_node_body(x_ref, hx_ref, dp1_ref, dp2_ref, ap1_ref, ap2_ref,
                  gvec_ref, es1_ref, es2_ref, g_ref,
                  g1w1, g1b1, g1w2, g1b2,
                  g2w1, g2b1, g2w2, g2b2,
                  g3w1, g3b1, g3w2, g3b2,
                  xnew_ref, gnew_ref, xsum_ref):
    i = pl.program_id(0)
    d = (dp1_ref[0] + dp1_ref[1]) + (dp2_ref[0] + dp2_ref[1]) + 1e-7
    a = (ap1_ref[0] + ap1_ref[1]) + (ap2_ref[0] + ap2_ref[1])
    hn = hx_ref[...] + a / d + gvec_ref[1:2, :]
    xn = x_ref[...] + jnp.maximum(_inorm(hn), 0.0)
    xnew_ref[...] = xn

    @pl.when(i == 0)
    def _():
        xsum_ref[...] = jnp.zeros((8, D), F32)
        gnew_ref[...] = jnp.zeros((8, D), F32)

    xsum_ref[0:1, :] += jnp.sum(xn, axis=0, keepdims=True)

    @pl.when(i == (N // BN) - 1)
    def _():
        g = g_ref[...]
        xm = xsum_ref[0:1, :] * (1.0 / N)
        em = (es1_ref[0:1, :] + es2_ref[0:1, :]) * (1.0 / E)
        hg = (_mlp(xm, g1w1[...], g1b1[...], g1w2[...], g1b2[...])
              + _mlp(em, g2w1[...], g2b1[...], g2w2[...], g2b2[...])
              + _mlp(g, g3w1[...], g3b1[...], g3w2[...], g3b2[...]))
        gnew_ref[0:1, :] = g + jnp.maximum(_inorm(hg), 0.0)


def _tc_node(x, hx, dp1, dp2, ap1, ap2, gvec, es1, es2, g, wts):
    wspecs = [_full(w.shape) for w in wts]
    part = pl.BlockSpec((NC, BN, D), lambda i: (0, i, 0))
    return pl.pallas_call(
        _tc_node_body,
        grid=(N // BN,),
        in_specs=[pl.BlockSpec((BN, D), lambda i: (i, 0)),
                  pl.BlockSpec((BN, D), lambda i: (i, 0)),
                  part, part, part, part,
                  _full((8, D)), _full((8, D)), _full((8, D)),
                  _full((1, D))] + wspecs,
        out_specs=[pl.BlockSpec((BN, D), lambda i: (i, 0)),
                   _full((8, D))],
        out_shape=[jax.ShapeDtypeStruct((N, D), F32),
                   jax.ShapeDtypeStruct((8, D), F32)],
        scratch_shapes=[pltpu.VMEM((8, D), F32)],
    )(x, hx, dp1, dp2, ap1, ap2, gvec, es1, es2, g, *wts)


def _wt(p):
    w1, b1, w2, b2 = p
    return (w1.T.astype(BF16), b1.reshape(1, H),
            w2.T.astype(BF16), b2.reshape(1, D))


def _idx3(v, nch, chunk):
    v3 = v.reshape(NT, nch, chunk)
    pad = jnp.zeros((NT, CPAD - nch, chunk), jnp.int32)
    return jnp.concatenate([v3, pad], axis=1)


def _idx3d(v):
    v3 = v.reshape(NT, DNCH, DCH)
    pad = jnp.zeros((NT, CPAD - DNCH, DCH), jnp.int32)
    return jnp.concatenate([v3, pad], axis=1)


def kernel(node_features, edge_index, edge_features, global_features, params):
    x = node_features[0]
    e = edge_features[0]
    g = global_features[0]
    src = edge_index[0, 0]
    dst = edge_index[0, 1]
    p = params

    pre_wts = _wt(p["node1"]) + _wt(p["node2"]) + _wt(p["edge3"]) + _wt(p["node3"])
    msg, hx, gvec = _tc_pre(x, g, pre_wts)

    edge_wts = _wt(p["edge1"]) + _wt(p["edge2"])
    zeros_nd = jnp.zeros((N, D), F32)

    results = []
    for h in (0, 1):
        src_h = lax.slice_in_dim(src, h * EH, (h + 1) * EH)
        dst_h = lax.slice_in_dim(dst, h * EH, (h + 1) * EH)
        sd3 = jnp.concatenate([src_h, dst_h]).reshape(NT, ANCH, ACH)
        src3 = _idx3(src_h, CNCH, CCH)
        dst3 = _idx3(dst_h, CNCH, CCH)

        xsd = _sc_gather(x, sd3)
        e_new_h, s_h, esum_h = _tc_edge(xsd, e, gvec, edge_wts, h)
        dpart_h = _sc_denom(s_h, src3, dst3, zeros_nd)
        apart_h = _sc_agg(s_h, src3, dst3, msg, zeros_nd)
        results.append((e_new_h, esum_h, dpart_h, apart_h))

    (e_new1, es1, dp1, ap1), (e_new2, es2, dp2, ap2) = results

    glob_wts = _wt(p["glob1"]) + _wt(p["glob2"]) + _wt(p["glob3"])
    x_new, g8 = _tc_node(x, hx, dp1, dp2, ap1, ap2, gvec, es1, es2, g, glob_wts)

    e_new = jnp.concatenate([e_new1, e_new2], axis=0)
    return (x_new[None], e_new[None], g8[0:1][None])
